# baseline (speedup 1.0000x reference)
import functools

import jax
import jax.numpy as jnp
import numpy as np
from jax import lax
from jax.experimental import pallas as pl
from jax.experimental.pallas import tpu as pltpu
from jax.experimental.pallas import tpu_sc as plsc

K_TOP = 3
T_LEN = 2048
PMAX = 64
PMIN = 32
NBINS = T_LEN // 2
CMAX = T_LEN // PMIN
SEQ_PER_W = 16


def _dft_basis():
    t = np.arange(T_LEN, dtype=np.int64)
    k = np.arange(1, NBINS + 1, dtype=np.int64)
    phase = 2.0 * np.pi * ((np.outer(t, k) % T_LEN) / float(T_LEN))
    w = np.concatenate([np.cos(phase), np.sin(phase)], axis=1).astype(np.float32)
    w_hi = jnp.asarray(w).astype(jnp.bfloat16)
    w_lo = (jnp.asarray(w) - w_hi.astype(jnp.float32)).astype(jnp.bfloat16)
    return w_hi, w_lo


def _spec_body(x_ref, whi_ref, wlo_ref, kamp_ref, meta_ref):
    xb = x_ref[...]
    x_hi = xb.astype(jnp.bfloat16)
    x_lo = (xb - x_hi.astype(jnp.float32)).astype(jnp.bfloat16)
    dims = (((1,), (0,)), ((), ()))
    whi = whi_ref[...]
    prod = jax.lax.dot_general(
        x_hi, whi, dims, preferred_element_type=jnp.float32)
    prod += jax.lax.dot_general(
        x_hi, wlo_ref[...], dims, preferred_element_type=jnp.float32)
    prod += jax.lax.dot_general(
        x_lo, whi, dims, preferred_element_type=jnp.float32)
    re = prod[:, :NBINS]
    im = prod[:, NBINS:]
    amp2 = re * re + im * im
    rows = amp2.shape[0]
    lane = jax.lax.broadcasted_iota(jnp.int32, (rows, NBINS), 1)
    vals, idxs = [], []
    a = amp2
    for _ in range(K_TOP):
        m = jnp.max(a, axis=-1, keepdims=True)
        i = jnp.min(jnp.where(a == m, lane, NBINS * 2), axis=-1, keepdims=True)
        vals.append(m)
        idxs.append(i)
        a = jnp.where(lane == i, -1.0, a)
    lane128 = jax.lax.broadcasted_iota(jnp.int32, (rows, 128), 1)
    lane256 = jax.lax.broadcasted_iota(jnp.int32, (rows, 256), 1)
    kamp = jnp.zeros((rows, 128), jnp.float32)
    meta = jnp.zeros((rows, 256), jnp.int32)
    for k in range(K_TOP):
        kidx = idxs[k] + 1
        amp_k = jnp.sqrt(vals[k])
        pf = jnp.floor(float(T_LEN) / kidx.astype(jnp.float32))
        p = jnp.clip(pf.astype(jnp.int32), PMIN, PMAX)
        cyc = jnp.floor(float(T_LEN) / p.astype(jnp.float32)).astype(jnp.int32)
        base = T_LEN - cyc * p
        kamp = jnp.where(lane128 == k, amp_k, kamp)
        grp = lane256 - 48 * k
        meta = jnp.where((grp >= 0) & (grp < 16), p, meta)
        meta = jnp.where((grp >= 16) & (grp < 32), base, meta)
        meta = jnp.where((grp >= 32) & (grp < 48), cyc, meta)
    kamp_ref[...] = kamp
    meta_ref[...] = meta


def _spectrum_topk(seqs):
    bn = seqs.shape[0]
    return pl.pallas_call(
        _spec_body,
        grid=(1,),
        in_specs=[
            pl.BlockSpec((bn, T_LEN), lambda i: (0, 0)),
            pl.BlockSpec((T_LEN, 2 * NBINS), lambda i: (0, 0)),
            pl.BlockSpec((T_LEN, 2 * NBINS), lambda i: (0, 0)),
        ],
        out_specs=[
            pl.BlockSpec((bn, 128), lambda i: (0, 0)),
            pl.BlockSpec((bn, 256), lambda i: (0, 0)),
        ],
        out_shape=[
            jax.ShapeDtypeStruct((bn, 128), jnp.float32),
            jax.ShapeDtypeStruct((bn, 256), jnp.int32),
        ],
    )(seqs, *_dft_basis())


def _mask_body(meta_ref, mask_ref):
    rows = meta_ref.shape[0]
    c_i = jax.lax.broadcasted_iota(jnp.int32, (rows, CMAX, PMAX), 1)
    p_i = jax.lax.broadcasted_iota(jnp.int32, (rows, CMAX, PMAX), 2)
    for k in range(K_TOP):
        p = meta_ref[:, 48 * k:48 * k + 1]
        cyc = meta_ref[:, 48 * k + 32:48 * k + 33]
        m = (p_i < p[:, :, None]) & (c_i < cyc[:, :, None])
        mask_ref[:, k] = m.astype(jnp.float32)


def _mask_build(meta):
    bn = meta.shape[0]
    blk = 64
    return pl.pallas_call(
        _mask_body,
        grid=(bn // blk,),
        in_specs=[pl.BlockSpec((blk, 256), lambda i: (i, 0))],
        out_specs=pl.BlockSpec((blk, K_TOP, CMAX, PMAX), lambda i: (i, 0, 0, 0)),
        out_shape=jax.ShapeDtypeStruct((bn, K_TOP, CMAX, PMAX), jnp.float32),
    )(meta)


def _make_fold(bn):
    vlen = SEQ_PER_W * T_LEN
    vmax = vlen - 1
    mesh = plsc.VectorSubcoreMesh(core_axis_name="c", subcore_axis_name="s")

    @functools.partial(
        pl.kernel,
        mesh=mesh,
        compiler_params=pltpu.CompilerParams(needs_layout_passes=False),
        out_type=jax.ShapeDtypeStruct((bn, K_TOP, CMAX, PMAX), jnp.float32),
        scratch_types=[
            pltpu.VMEM((vlen,), jnp.float32),
            pltpu.VMEM((SEQ_PER_W * 256,), jnp.int32),
            pltpu.VMEM((CMAX, PMAX), jnp.float32),
            pltpu.VMEM((CMAX, PMAX), jnp.float32),
            pltpu.VMEM((CMAX, PMAX), jnp.float32),
            pltpu.SemaphoreType.DMA,
            pltpu.SemaphoreType.DMA,
            pltpu.SemaphoreType.DMA,
        ],
    )
    def fold(seqs_hbm, meta_hbm, gat_hbm, seqs_v, meta_v, g0, g1, g2,
             sem0, sem1, sem2):
        wid = lax.axis_index("s") * 2 + lax.axis_index("c")
        iota16 = lax.iota(jnp.int32, 16)
        pltpu.sync_copy(seqs_hbm.at[pl.ds(wid * vlen, vlen)], seqs_v)
        pltpu.sync_copy(
            meta_hbm.at[pl.ds(wid * SEQ_PER_W * 256, SEQ_PER_W * 256)], meta_v)
        gbufs = (g0, g1, g2)
        sems = (sem0, sem1, sem2)

        zero16 = jnp.zeros((16,), jnp.float32)

        def seq_body(t, carry):
            s = wid * SEQ_PER_W + t
            tbase = t * T_LEN
            copies = []
            for k in range(K_TOP):
                moff = t * 256 + 48 * k
                pv = meta_v[pl.ds(moff, 16)]
                basev = meta_v[pl.ds(moff + 16, 16)] + tbase
                cycv = meta_v[pl.ds(moff + 32, 16)]
                p_s = jnp.max(pv)
                cyc_s = jnp.max(cycv)
                gv = gbufs[k]

                def tile(nf, pv=pv, basev=basev, cyc_s=cyc_s, gv=gv):
                    if nf < 4:
                        cb = nf * 16 + iota16
                        cbc = jnp.minimum(cb, pv - 1)
                        mfb = jnp.where(cb < pv, 1.0, 0.0).astype(jnp.float32)

                    @plsc.parallel_loop(0, cyc_s, unroll=4)
                    def c_body(c):
                        bc = basev + c * pv
                        for j in range(nf):
                            gv[c, pl.ds(j * 16, 16)] = plsc.load_gather(
                                seqs_v, [bc + (j * 16 + iota16)])
                        if nf < 4:
                            gv[c, pl.ds(nf * 16, 16)] = plsc.load_gather(
                                seqs_v, [bc + cbc]) * mfb
                            for j in range(nf + 1, 4):
                                gv[c, pl.ds(j * 16, 16)] = zero16

                    @plsc.parallel_loop(cyc_s, CMAX, unroll=4)
                    def z_body(c):
                        for j in range(4):
                            gv[c, pl.ds(j * 16, 16)] = zero16

                lax.cond(
                    p_s >= 64,
                    lambda: tile(4),
                    lambda: lax.cond(
                        p_s >= 48, lambda: tile(3), lambda: tile(2)))
                copies.append(pltpu.async_copy(gv, gat_hbm.at[s, k], sems[k]))
            for cp in copies:
                cp.wait()
            return carry

        lax.fori_loop(0, SEQ_PER_W, seq_body, 0)

    return fold


def kernel(x):
    b, t, n = x.shape
    bn = b * n
    seqs3 = jnp.transpose(x, (0, 2, 1))
    seqs = seqs3.reshape(bn, t)
    kamp128, meta = _spectrum_topk(seqs)
    flat_mask = _mask_build(meta).reshape(b, n, K_TOP, CMAX, PMAX)
    gat = _make_fold(bn)(seqs3.reshape(-1), meta.reshape(-1))
    gathered = gat.reshape(b, n, K_TOP, CMAX, PMAX)
    kamp = kamp128[:, :K_TOP].reshape(b, n, K_TOP)
    return gathered, flat_mask, kamp


# speedup vs baseline: 1.0028x; 1.0028x over previous
import functools

import jax
import jax.numpy as jnp
import numpy as np
from jax import lax
from jax.experimental import pallas as pl
from jax.experimental.pallas import tpu as pltpu
from jax.experimental.pallas import tpu_sc as plsc

K_TOP = 3
T_LEN = 2048
PMAX = 64
PMIN = 32
NBINS = T_LEN // 2
CMAX = T_LEN // PMIN
SEQ_PER_W = 16


def _dft_basis():
    t = np.arange(T_LEN, dtype=np.int64)
    k = np.arange(1, NBINS + 1, dtype=np.int64)
    phase = 2.0 * np.pi * ((np.outer(t, k) % T_LEN) / float(T_LEN))
    w = np.concatenate([np.cos(phase), np.sin(phase)], axis=1).astype(np.float32)
    w_hi = jnp.asarray(w).astype(jnp.bfloat16)
    w_lo = (jnp.asarray(w) - w_hi.astype(jnp.float32)).astype(jnp.bfloat16)
    return jnp.concatenate([w_hi, w_lo, w_hi], axis=0)


def _spec_body(x_ref, w_ref, kamp_ref, meta_ref):
    xb = x_ref[...]
    x_hi = xb.astype(jnp.bfloat16)
    x_lo = (xb - x_hi.astype(jnp.float32)).astype(jnp.bfloat16)
    xs = jnp.concatenate([x_hi, x_hi, x_lo], axis=1)
    dims = (((1,), (0,)), ((), ()))
    prod = jax.lax.dot_general(
        xs, w_ref[...], dims, preferred_element_type=jnp.float32)
    re = prod[:, :NBINS]
    im = prod[:, NBINS:]
    amp2 = re * re + im * im
    rows = amp2.shape[0]
    lane = jax.lax.broadcasted_iota(jnp.int32, (rows, NBINS), 1)
    vals, idxs = [], []
    a = amp2
    for _ in range(K_TOP):
        m = jnp.max(a, axis=-1, keepdims=True)
        i = jnp.min(jnp.where(a == m, lane, NBINS * 2), axis=-1, keepdims=True)
        vals.append(m)
        idxs.append(i)
        a = jnp.where(lane == i, -1.0, a)
    lane128 = jax.lax.broadcasted_iota(jnp.int32, (rows, 128), 1)
    lane256 = jax.lax.broadcasted_iota(jnp.int32, (rows, 256), 1)
    kamp = jnp.zeros((rows, 128), jnp.float32)
    meta = jnp.zeros((rows, 256), jnp.int32)
    for k in range(K_TOP):
        kidx = idxs[k] + 1
        amp_k = jnp.sqrt(vals[k])
        pf = jnp.floor(float(T_LEN) / kidx.astype(jnp.float32))
        p = jnp.clip(pf.astype(jnp.int32), PMIN, PMAX)
        cyc = jnp.floor(float(T_LEN) / p.astype(jnp.float32)).astype(jnp.int32)
        base = T_LEN - cyc * p
        kamp = jnp.where(lane128 == k, amp_k, kamp)
        grp = lane256 - 48 * k
        meta = jnp.where((grp >= 0) & (grp < 16), p, meta)
        meta = jnp.where((grp >= 16) & (grp < 32), base, meta)
        meta = jnp.where((grp >= 32) & (grp < 48), cyc, meta)
    kamp_ref[...] = kamp
    meta_ref[...] = meta


def _spectrum_topk(seqs):
    bn = seqs.shape[0]
    return pl.pallas_call(
        _spec_body,
        grid=(1,),
        in_specs=[
            pl.BlockSpec((bn, T_LEN), lambda i: (0, 0)),
            pl.BlockSpec((3 * T_LEN, 2 * NBINS), lambda i: (0, 0)),
        ],
        out_specs=[
            pl.BlockSpec((bn, 128), lambda i: (0, 0)),
            pl.BlockSpec((bn, 256), lambda i: (0, 0)),
        ],
        out_shape=[
            jax.ShapeDtypeStruct((bn, 128), jnp.float32),
            jax.ShapeDtypeStruct((bn, 256), jnp.int32),
        ],
    )(seqs, _dft_basis())


def _mask_body(meta_ref, mask_ref):
    rows = meta_ref.shape[0]
    c_i = jax.lax.broadcasted_iota(jnp.int32, (rows, CMAX, PMAX), 1)
    p_i = jax.lax.broadcasted_iota(jnp.int32, (rows, CMAX, PMAX), 2)
    for k in range(K_TOP):
        p = meta_ref[:, 48 * k:48 * k + 1]
        cyc = meta_ref[:, 48 * k + 32:48 * k + 33]
        m = (p_i < p[:, :, None]) & (c_i < cyc[:, :, None])
        mask_ref[:, k] = m.astype(jnp.float32)


def _mask_build(meta):
    bn = meta.shape[0]
    blk = 64
    return pl.pallas_call(
        _mask_body,
        grid=(bn // blk,),
        in_specs=[pl.BlockSpec((blk, 256), lambda i: (i, 0))],
        out_specs=pl.BlockSpec((blk, K_TOP, CMAX, PMAX), lambda i: (i, 0, 0, 0)),
        out_shape=jax.ShapeDtypeStruct((bn, K_TOP, CMAX, PMAX), jnp.float32),
    )(meta)


def _make_fold(bn):
    vlen = SEQ_PER_W * T_LEN
    vmax = vlen - 1
    mesh = plsc.VectorSubcoreMesh(core_axis_name="c", subcore_axis_name="s")

    @functools.partial(
        pl.kernel,
        mesh=mesh,
        compiler_params=pltpu.CompilerParams(needs_layout_passes=False),
        out_type=jax.ShapeDtypeStruct((bn, K_TOP, CMAX, PMAX), jnp.float32),
        scratch_types=[
            pltpu.VMEM((vlen,), jnp.float32),
            pltpu.VMEM((SEQ_PER_W * 256,), jnp.int32),
            pltpu.VMEM((CMAX, PMAX), jnp.float32),
            pltpu.VMEM((CMAX, PMAX), jnp.float32),
            pltpu.VMEM((CMAX, PMAX), jnp.float32),
            pltpu.SemaphoreType.DMA,
            pltpu.SemaphoreType.DMA,
            pltpu.SemaphoreType.DMA,
        ],
    )
    def fold(seqs_hbm, meta_hbm, gat_hbm, seqs_v, meta_v, g0, g1, g2,
             sem0, sem1, sem2):
        wid = lax.axis_index("s") * 2 + lax.axis_index("c")
        iota16 = lax.iota(jnp.int32, 16)
        pltpu.sync_copy(seqs_hbm.at[pl.ds(wid * vlen, vlen)], seqs_v)
        pltpu.sync_copy(
            meta_hbm.at[pl.ds(wid * SEQ_PER_W * 256, SEQ_PER_W * 256)], meta_v)
        gbufs = (g0, g1, g2)
        sems = (sem0, sem1, sem2)

        zero16 = jnp.zeros((16,), jnp.float32)

        def seq_body(t, carry):
            s = wid * SEQ_PER_W + t
            tbase = t * T_LEN
            copies = []
            for k in range(K_TOP):
                moff = t * 256 + 48 * k
                pv = meta_v[pl.ds(moff, 16)]
                basev = meta_v[pl.ds(moff + 16, 16)] + tbase
                cycv = meta_v[pl.ds(moff + 32, 16)]
                p_s = jnp.max(pv)
                cyc_s = jnp.max(cycv)
                gv = gbufs[k]

                def tile(nf, pv=pv, basev=basev, cyc_s=cyc_s, gv=gv):
                    if nf < 4:
                        cb = nf * 16 + iota16
                        cbc = jnp.minimum(cb, pv - 1)
                        mfb = jnp.where(cb < pv, 1.0, 0.0).astype(jnp.float32)

                    @plsc.parallel_loop(0, cyc_s, unroll=2)
                    def c_body(c):
                        bc = basev + c * pv
                        for j in range(nf):
                            gv[c, pl.ds(j * 16, 16)] = plsc.load_gather(
                                seqs_v, [bc + (j * 16 + iota16)])
                        if nf < 4:
                            gv[c, pl.ds(nf * 16, 16)] = plsc.load_gather(
                                seqs_v, [bc + cbc]) * mfb
                            for j in range(nf + 1, 4):
                                gv[c, pl.ds(j * 16, 16)] = zero16

                    @plsc.parallel_loop(cyc_s, CMAX, unroll=2)
                    def z_body(c):
                        for j in range(4):
                            gv[c, pl.ds(j * 16, 16)] = zero16

                lax.cond(
                    p_s >= 64,
                    lambda: tile(4),
                    lambda: lax.cond(
                        p_s >= 48, lambda: tile(3), lambda: tile(2)))
                copies.append(pltpu.async_copy(gv, gat_hbm.at[s, k], sems[k]))
            for cp in copies:
                cp.wait()
            return carry

        lax.fori_loop(0, SEQ_PER_W, seq_body, 0)

    return fold


def kernel(x):
    b, t, n = x.shape
    bn = b * n
    seqs3 = jnp.transpose(x, (0, 2, 1))
    seqs = seqs3.reshape(bn, t)
    kamp128, meta = _spectrum_topk(seqs)
    flat_mask = _mask_build(meta).reshape(b, n, K_TOP, CMAX, PMAX)
    gat = _make_fold(bn)(seqs3.reshape(-1), meta.reshape(-1))
    gathered = gat.reshape(b, n, K_TOP, CMAX, PMAX)
    kamp = kamp128[:, :K_TOP].reshape(b, n, K_TOP)
    return gathered, flat_mask, kamp


# speedup vs baseline: 1.0394x; 1.0365x over previous
import functools

import jax
import jax.numpy as jnp
import numpy as np
from jax import lax
from jax.experimental import pallas as pl
from jax.experimental.pallas import tpu as pltpu
from jax.experimental.pallas import tpu_sc as plsc

K_TOP = 3
T_LEN = 2048
PMAX = 64
PMIN = 32
NBINS = T_LEN // 2
CMAX = T_LEN // PMIN
SEQ_PER_W = 16


def _dft_basis():
    t = np.arange(T_LEN, dtype=np.int64)
    k = np.arange(1, NBINS + 1, dtype=np.int64)
    phase = 2.0 * np.pi * ((np.outer(t, k) % T_LEN) / float(T_LEN))
    w = np.concatenate([np.cos(phase), np.sin(phase)], axis=1).astype(np.float32)
    w_hi = jnp.asarray(w).astype(jnp.bfloat16)
    w_lo = (jnp.asarray(w) - w_hi.astype(jnp.float32)).astype(jnp.bfloat16)
    return w_hi, w_lo


def _spec_body(x_ref, whi_ref, wlo_ref, kamp_ref, meta_ref):
    xb = x_ref[...]
    x_hi = xb.astype(jnp.bfloat16)
    x_lo = (xb - x_hi.astype(jnp.float32)).astype(jnp.bfloat16)
    dims = (((1,), (0,)), ((), ()))
    whi = whi_ref[...]
    prod = jax.lax.dot_general(
        x_hi, whi, dims, preferred_element_type=jnp.float32)
    prod += jax.lax.dot_general(
        x_hi, wlo_ref[...], dims, preferred_element_type=jnp.float32)
    prod += jax.lax.dot_general(
        x_lo, whi, dims, preferred_element_type=jnp.float32)
    re = prod[:, :NBINS]
    im = prod[:, NBINS:]
    amp2 = re * re + im * im
    rows = amp2.shape[0]
    lane = jax.lax.broadcasted_iota(jnp.int32, (rows, NBINS), 1)
    vals, idxs = [], []
    a = amp2
    for _ in range(K_TOP):
        m = jnp.max(a, axis=-1, keepdims=True)
        i = jnp.min(jnp.where(a == m, lane, NBINS * 2), axis=-1, keepdims=True)
        vals.append(m)
        idxs.append(i)
        a = jnp.where(lane == i, -1.0, a)
    lane128 = jax.lax.broadcasted_iota(jnp.int32, (rows, 128), 1)
    lane256 = jax.lax.broadcasted_iota(jnp.int32, (rows, 256), 1)
    kamp = jnp.zeros((rows, 128), jnp.float32)
    meta = jnp.zeros((rows, 256), jnp.int32)
    for k in range(K_TOP):
        kidx = idxs[k] + 1
        amp_k = jnp.sqrt(vals[k])
        pf = jnp.floor(float(T_LEN) / kidx.astype(jnp.float32))
        p = jnp.clip(pf.astype(jnp.int32), PMIN, PMAX)
        cyc = jnp.floor(float(T_LEN) / p.astype(jnp.float32)).astype(jnp.int32)
        base = T_LEN - cyc * p
        kamp = jnp.where(lane128 == k, amp_k, kamp)
        grp = lane256 - 48 * k
        meta = jnp.where((grp >= 0) & (grp < 16), p, meta)
        meta = jnp.where((grp >= 16) & (grp < 32), base, meta)
        meta = jnp.where((grp >= 32) & (grp < 48), cyc, meta)
    kamp_ref[...] = kamp
    meta_ref[...] = meta


def _spectrum_topk(seqs):
    bn = seqs.shape[0]
    return pl.pallas_call(
        _spec_body,
        grid=(1,),
        in_specs=[
            pl.BlockSpec((bn, T_LEN), lambda i: (0, 0)),
            pl.BlockSpec((T_LEN, 2 * NBINS), lambda i: (0, 0)),
            pl.BlockSpec((T_LEN, 2 * NBINS), lambda i: (0, 0)),
        ],
        out_specs=[
            pl.BlockSpec((bn, 128), lambda i: (0, 0)),
            pl.BlockSpec((bn, 256), lambda i: (0, 0)),
        ],
        out_shape=[
            jax.ShapeDtypeStruct((bn, 128), jnp.float32),
            jax.ShapeDtypeStruct((bn, 256), jnp.int32),
        ],
    )(seqs, *_dft_basis())


def _mask_body(meta_ref, mask_ref):
    rows = meta_ref.shape[0]
    c_i = jax.lax.broadcasted_iota(jnp.int32, (rows, CMAX, PMAX), 1)
    p_i = jax.lax.broadcasted_iota(jnp.int32, (rows, CMAX, PMAX), 2)
    for k in range(K_TOP):
        p = meta_ref[:, 48 * k:48 * k + 1]
        cyc = meta_ref[:, 48 * k + 32:48 * k + 33]
        m = (p_i < p[:, :, None]) & (c_i < cyc[:, :, None])
        mask_ref[:, k] = m.astype(jnp.float32)


def _mask_build(meta):
    bn = meta.shape[0]
    blk = 64
    return pl.pallas_call(
        _mask_body,
        grid=(bn // blk,),
        in_specs=[pl.BlockSpec((blk, 256), lambda i: (i, 0))],
        out_specs=pl.BlockSpec((blk, K_TOP, CMAX, PMAX), lambda i: (i, 0, 0, 0)),
        out_shape=jax.ShapeDtypeStruct((bn, K_TOP, CMAX, PMAX), jnp.float32),
    )(meta)


def _make_fold(bn):
    vlen = SEQ_PER_W * T_LEN
    vmax = vlen - 1
    mesh = plsc.VectorSubcoreMesh(core_axis_name="c", subcore_axis_name="s")

    @functools.partial(
        pl.kernel,
        mesh=mesh,
        compiler_params=pltpu.CompilerParams(needs_layout_passes=False),
        out_type=jax.ShapeDtypeStruct((bn, K_TOP, CMAX, PMAX), jnp.float32),
        scratch_types=[
            pltpu.VMEM((vlen,), jnp.float32),
            pltpu.VMEM((SEQ_PER_W * 256,), jnp.int32),
            pltpu.VMEM((CMAX, PMAX), jnp.float32),
            pltpu.VMEM((CMAX, PMAX), jnp.float32),
            pltpu.VMEM((CMAX, PMAX), jnp.float32),
            pltpu.SemaphoreType.DMA,
            pltpu.SemaphoreType.DMA,
            pltpu.SemaphoreType.DMA,
        ],
    )
    def fold(seqs_hbm, meta_hbm, gat_hbm, seqs_v, meta_v, g0, g1, g2,
             sem0, sem1, sem2):
        wid = lax.axis_index("s") * 2 + lax.axis_index("c")
        iota16 = lax.iota(jnp.int32, 16)
        pltpu.sync_copy(seqs_hbm.at[pl.ds(wid * vlen, vlen)], seqs_v)
        pltpu.sync_copy(
            meta_hbm.at[pl.ds(wid * SEQ_PER_W * 256, SEQ_PER_W * 256)], meta_v)
        gbufs = (g0, g1, g2)
        sems = (sem0, sem1, sem2)

        zero16 = jnp.zeros((16,), jnp.float32)

        def seq_body(t, carry):
            s = wid * SEQ_PER_W + t
            tbase = t * T_LEN
            for k in range(K_TOP):
                moff = t * 256 + 48 * k
                pv = meta_v[pl.ds(moff, 16)]
                basev = meta_v[pl.ds(moff + 16, 16)] + tbase
                cycv = meta_v[pl.ds(moff + 32, 16)]
                p_s = jnp.max(pv)
                cyc_s = jnp.max(cycv)
                gv = gbufs[k]

                @pl.when(t > 0)
                def _wait_prev(k=k, s=s):
                    pltpu.make_async_copy(
                        gbufs[k], gat_hbm.at[s - 1, k], sems[k]).wait()

                def tile(nf, pv=pv, basev=basev, cyc_s=cyc_s, gv=gv):
                    if nf < 4:
                        cb = nf * 16 + iota16
                        cbc = jnp.minimum(cb, pv - 1)
                        mfb = jnp.where(cb < pv, 1.0, 0.0).astype(jnp.float32)

                    @plsc.parallel_loop(0, cyc_s, unroll=2)
                    def c_body(c):
                        bc = basev + c * pv
                        for j in range(nf):
                            gv[c, pl.ds(j * 16, 16)] = plsc.load_gather(
                                seqs_v, [bc + (j * 16 + iota16)])
                        if nf < 4:
                            gv[c, pl.ds(nf * 16, 16)] = plsc.load_gather(
                                seqs_v, [bc + cbc]) * mfb
                            for j in range(nf + 1, 4):
                                gv[c, pl.ds(j * 16, 16)] = zero16

                    @plsc.parallel_loop(cyc_s, CMAX, unroll=2)
                    def z_body(c):
                        for j in range(4):
                            gv[c, pl.ds(j * 16, 16)] = zero16

                lax.cond(
                    p_s >= 64,
                    lambda: tile(4),
                    lambda: lax.cond(
                        p_s >= 48, lambda: tile(3), lambda: tile(2)))
                pltpu.async_copy(gv, gat_hbm.at[s, k], sems[k])
            return carry

        lax.fori_loop(0, SEQ_PER_W, seq_body, 0)
        last = wid * SEQ_PER_W + SEQ_PER_W - 1
        for k in range(K_TOP):
            pltpu.make_async_copy(gbufs[k], gat_hbm.at[last, k], sems[k]).wait()

    return fold


def kernel(x):
    b, t, n = x.shape
    bn = b * n
    seqs3 = jnp.transpose(x, (0, 2, 1))
    seqs = seqs3.reshape(bn, t)
    kamp128, meta = _spectrum_topk(seqs)
    flat_mask = _mask_build(meta).reshape(b, n, K_TOP, CMAX, PMAX)
    gat = _make_fold(bn)(seqs3.reshape(-1), meta.reshape(-1))
    gathered = gat.reshape(b, n, K_TOP, CMAX, PMAX)
    kamp = kamp128[:, :K_TOP].reshape(b, n, K_TOP)
    return gathered, flat_mask, kamp
